# baseline (speedup 1.0000x reference)
import functools

import jax
import jax.numpy as jnp
from jax import lax
from jax.experimental import pallas as pl
from jax.experimental.pallas import tpu as pltpu
from jax.experimental.pallas import tpu_sc as plsc

B = 1024
S = 51
SP = 56
D = 128
NW = 32
BPW = B // NW
MPAD = 64
NBUF = 4
L = 16


def _build():
    info = plsc.get_sparse_core_info()
    nc = info.num_cores
    mesh = plsc.VectorSubcoreMesh(core_axis_name="c", subcore_axis_name="s")

    @functools.partial(
        pl.kernel,
        mesh=mesh,
        out_type=jax.ShapeDtypeStruct((B, SP, D), jnp.float32),
        scratch_types=[
            pltpu.VMEM((BPW, SP), jnp.int32),
            pltpu.VMEM((BPW * MPAD,), jnp.int32),
            pltpu.VMEM((NBUF, SP, D), jnp.float32),
        ]
        + [pltpu.SemaphoreType.DMA] * (2 * NBUF),
    )
    def k(table_hbm, idx_hbm, msk_hbm, out_hbm, idx_v, msk_v, rows_v, *sems):
        gsem = sems[:NBUF]
        ssem = sems[NBUF:]
        wid = lax.axis_index("s") * nc + lax.axis_index("c")
        pltpu.sync_copy(idx_hbm.at[wid], idx_v)
        for b in range(NBUF):
            pltpu.async_copy(table_hbm.at[idx_v.at[b]], rows_v.at[b], gsem[b])
        pltpu.sync_copy(msk_hbm.at[pl.ds(wid * BPW * MPAD, BPW * MPAD)], msk_v)
        zeros = jnp.zeros((L,), jnp.float32)

        def body(p, carry):
            for b in range(NBUF):
                bl = p * NBUF + b
                prev = (b - 1) % NBUF

                @pl.when(jnp.logical_and(bl >= 1, bl - 1 + NBUF < BPW))
                def _(b=b, bl=bl, prev=prev):
                    with jax.named_scope("refill"):
                        pltpu.make_async_copy(
                            rows_v.at[prev],
                            out_hbm.at[wid * BPW + bl - 1],
                            ssem[prev],
                        ).wait()
                        pltpu.async_copy(
                            table_hbm.at[idx_v.at[bl - 1 + NBUF]],
                            rows_v.at[prev],
                            gsem[prev],
                        )

                with jax.named_scope("gwait"):
                    pltpu.make_async_copy(
                        table_hbm.at[idx_v.at[bl]],
                        rows_v.at[b],
                        gsem[b],
                    ).wait()

                with jax.named_scope("maskz"):
                    for g in range(4):
                        nt = L if g < 3 else S - 3 * L
                        mv = msk_v[pl.ds(bl * MPAD + g * L, L)]
                        for t in range(nt):
                            r = g * L + t

                            @pl.when(mv[t] != 0)
                            def _(r=r, b=b):
                                for j in range(D // L):
                                    rows_v[b, r, pl.ds(j * L, L)] = zeros

                with jax.named_scope("sstart"):
                    pltpu.async_copy(
                        rows_v.at[b], out_hbm.at[wid * BPW + bl], ssem[b]
                    )

            return carry

        lax.fori_loop(0, BPW // NBUF, body, 0)
        for b in range(NBUF):
            pltpu.make_async_copy(
                rows_v.at[b],
                out_hbm.at[wid * BPW + BPW - NBUF + b],
                ssem[b],
            ).wait()

    return k


_k = jax.jit(_build())


def kernel(node_ids, mask, emb_table):
    idx = jnp.pad(node_ids.astype(jnp.int32), ((0, 0), (0, SP - S))).reshape(NW, BPW, SP)
    msk = jnp.pad(mask.astype(jnp.int32), ((0, 0), (0, MPAD - S))).reshape(-1)
    return _k(emb_table, idx, msk)[:, :S, :]


# speedup vs baseline: 3.9746x; 3.9746x over previous
import functools

import jax
import jax.numpy as jnp
from jax import lax
from jax.experimental import pallas as pl
from jax.experimental.pallas import tpu as pltpu
from jax.experimental.pallas import tpu_sc as plsc

B = 1024
S = 51
SP = 56
D = 128
NW = 32
BPW = B // NW
MPAD = 64
NBUF = 4
L = 16


def _build():
    info = plsc.get_sparse_core_info()
    nc = info.num_cores
    mesh = plsc.VectorSubcoreMesh(core_axis_name="c", subcore_axis_name="s")

    @functools.partial(
        pl.kernel,
        mesh=mesh,
        out_type=jax.ShapeDtypeStruct((B, SP, D), jnp.float32),
        scratch_types=[
            pltpu.VMEM((BPW, SP), jnp.int32),
            pltpu.VMEM((BPW * MPAD,), jnp.int32),
            pltpu.VMEM((NBUF, SP, D), jnp.float32),
        ]
        + [pltpu.SemaphoreType.DMA] * (2 * NBUF),
    )
    def k(table_hbm, idx_hbm, msk_hbm, out_hbm, idx_v, msk_v, rows_v, *sems):
        gsem = sems[:NBUF]
        ssem = sems[NBUF:]
        wid = lax.axis_index("s") * nc + lax.axis_index("c")
        pltpu.sync_copy(idx_hbm.at[wid], idx_v)
        for b in range(NBUF):
            pltpu.async_copy(table_hbm.at[idx_v.at[b]], rows_v.at[b], gsem[b])
        pltpu.sync_copy(msk_hbm.at[pl.ds(wid * BPW * MPAD, BPW * MPAD)], msk_v)
        zeros = jnp.zeros((L,), jnp.float32)

        def body(p, carry):
            for b in range(NBUF):
                bl = p * NBUF + b
                prev = (b - 1) % NBUF

                @pl.when(jnp.logical_and(bl >= 1, bl - 1 + NBUF < BPW))
                def _(b=b, bl=bl, prev=prev):
                    with jax.named_scope("refill"):
                        pltpu.make_async_copy(
                            rows_v.at[prev],
                            out_hbm.at[wid * BPW + bl - 1],
                            ssem[prev],
                        ).wait()
                        pltpu.async_copy(
                            table_hbm.at[idx_v.at[bl - 1 + NBUF]],
                            rows_v.at[prev],
                            gsem[prev],
                        )

                with jax.named_scope("gwait"):
                    pltpu.make_async_copy(
                        table_hbm.at[idx_v.at[bl]],
                        rows_v.at[b],
                        gsem[b],
                    ).wait()

                with jax.named_scope("maskz"):
                    for g in range(4):
                        nt = L if g < 3 else S - 3 * L
                        mv = msk_v[pl.ds(bl * MPAD + g * L, L)]
                        for t in range(nt):
                            r = g * L + t

                            @pl.when(mv[t] != 0)
                            def _(r=r, b=b):
                                for j in range(D // L):
                                    rows_v[b, r, pl.ds(j * L, L)] = zeros

                with jax.named_scope("sstart"):
                    pltpu.async_copy(
                        rows_v.at[b], out_hbm.at[wid * BPW + bl], ssem[b]
                    )

            return carry

        lax.fori_loop(0, BPW // NBUF, body, 0)
        for b in range(NBUF):
            pltpu.make_async_copy(
                rows_v.at[b],
                out_hbm.at[wid * BPW + BPW - NBUF + b],
                ssem[b],
            ).wait()

    return k


_k = jax.jit(_build())


def kernel(node_ids, mask, emb_table):
    fill = (jnp.arange(B, dtype=jnp.int32)[:, None] * (SP - S)
            + jnp.arange(SP - S, dtype=jnp.int32)[None, :]) % (B * (SP - S))
    idx = jnp.concatenate([node_ids.astype(jnp.int32), fill], axis=1).reshape(NW, BPW, SP)
    msk = jnp.pad(mask.astype(jnp.int32), ((0, 0), (0, MPAD - S))).reshape(-1)
    return _k(emb_table, idx, msk)[:, :S, :]


# speedup vs baseline: 4.0758x; 1.0254x over previous
import functools

import jax
import jax.numpy as jnp
from jax import lax
from jax.experimental import pallas as pl
from jax.experimental.pallas import tpu as pltpu
from jax.experimental.pallas import tpu_sc as plsc

B = 1024
S = 51
SP = 56
D = 128
NW = 32
BPW = B // NW
MPAD = 64
NBUF = 4
L = 16


def _build():
    info = plsc.get_sparse_core_info()
    nc = info.num_cores
    mesh = plsc.VectorSubcoreMesh(core_axis_name="c", subcore_axis_name="s")

    @functools.partial(
        pl.kernel,
        mesh=mesh,
        out_type=jax.ShapeDtypeStruct((B, S, D), jnp.float32),
        scratch_types=[
            pltpu.VMEM((BPW, SP), jnp.int32),
            pltpu.VMEM((BPW * MPAD,), jnp.int32),
            pltpu.VMEM((NBUF, SP, D), jnp.float32),
        ]
        + [pltpu.SemaphoreType.DMA] * (2 * NBUF),
    )
    def k(table_hbm, idx_hbm, msk_hbm, out_hbm, idx_v, msk_v, rows_v, *sems):
        gsem = sems[:NBUF]
        ssem = sems[NBUF:]
        wid = lax.axis_index("s") * nc + lax.axis_index("c")
        pltpu.sync_copy(idx_hbm.at[wid], idx_v)
        for b in range(NBUF):
            pltpu.async_copy(table_hbm.at[idx_v.at[b]], rows_v.at[b], gsem[b])
        pltpu.sync_copy(msk_hbm.at[pl.ds(wid * BPW * MPAD, BPW * MPAD)], msk_v)
        zeros = jnp.zeros((L,), jnp.float32)

        def body(p, carry):
            for b in range(NBUF):
                bl = p * NBUF + b
                prev = (b - 1) % NBUF

                @pl.when(jnp.logical_and(bl >= 1, bl - 1 + NBUF < BPW))
                def _(b=b, bl=bl, prev=prev):
                    with jax.named_scope("refill"):
                        pltpu.make_async_copy(
                            rows_v.at[prev, pl.ds(0, S)],
                            out_hbm.at[wid * BPW + bl - 1],
                            ssem[prev],
                        ).wait()
                        pltpu.async_copy(
                            table_hbm.at[idx_v.at[bl - 1 + NBUF]],
                            rows_v.at[prev],
                            gsem[prev],
                        )

                with jax.named_scope("gwait"):
                    pltpu.make_async_copy(
                        table_hbm.at[idx_v.at[bl]],
                        rows_v.at[b],
                        gsem[b],
                    ).wait()

                with jax.named_scope("maskz"):
                    for g in range(4):
                        nt = L if g < 3 else S - 3 * L
                        mv = msk_v[pl.ds(bl * MPAD + g * L, L)]
                        for t in range(nt):
                            r = g * L + t

                            @pl.when(mv[t] != 0)
                            def _(r=r, b=b):
                                for j in range(D // L):
                                    rows_v[b, r, pl.ds(j * L, L)] = zeros

                with jax.named_scope("sstart"):
                    pltpu.async_copy(
                        rows_v.at[b, pl.ds(0, S)], out_hbm.at[wid * BPW + bl], ssem[b]
                    )

            return carry

        lax.fori_loop(0, BPW // NBUF, body, 0)
        for b in range(NBUF):
            pltpu.make_async_copy(
                rows_v.at[b, pl.ds(0, S)],
                out_hbm.at[wid * BPW + BPW - NBUF + b],
                ssem[b],
            ).wait()

    return k


_k = jax.jit(_build())


def kernel(node_ids, mask, emb_table):
    fill = (jnp.arange(B, dtype=jnp.int32)[:, None] * (SP - S)
            + jnp.arange(SP - S, dtype=jnp.int32)[None, :]) % (B * (SP - S))
    idx = jnp.concatenate([node_ids.astype(jnp.int32), fill], axis=1).reshape(NW, BPW, SP)
    msk = jnp.pad(mask.astype(jnp.int32), ((0, 0), (0, MPAD - S))).reshape(-1)
    return _k(emb_table, idx, msk)
